# baseline (speedup 1.0000x reference)
import jax
import jax.numpy as jnp
from jax import lax
from jax.experimental import pallas as pl
from jax.experimental.pallas import tpu as pltpu
from jax.experimental.pallas import tpu_sc as plsc

_NC = 2
_NS = 16
_NW = _NC * _NS
_NBUF = 2
_CHUNK = 32


def _copy_body(table_hbm, out_hbm, buf, in_sems, out_sems):
    wid = lax.axis_index("s") * _NC + lax.axis_index("c")
    rows = out_hbm.shape[0] // _NW
    nchunk = rows // _CHUNK
    base = wid * rows

    def in_copy(j, slot):
        return pltpu.make_async_copy(
            table_hbm.at[pl.ds(base + j * _CHUNK, _CHUNK)],
            buf.at[slot], in_sems.at[slot])

    def out_copy(j, slot):
        return pltpu.make_async_copy(
            buf.at[slot],
            out_hbm.at[pl.ds(base + j * _CHUNK, _CHUNK)], out_sems.at[slot])

    for s in range(_NBUF):
        in_copy(s, s).start()

    def group(g, carry):
        for b in range(_NBUF):
            j = g * _NBUF + b
            in_copy(j, b).wait()
            out_copy(j, b).start()
            out_copy(j, b).wait()

            @pl.when(j + _NBUF < nchunk)
            def _():
                in_copy(j + _NBUF, b).start()
        return carry

    lax.fori_loop(0, nchunk // _NBUF, group, 0)


def kernel(input_ids, pos_emb_table):
    seq_len = input_ids.shape[-1]
    emb = pos_emb_table.shape[1]
    mesh = plsc.VectorSubcoreMesh(core_axis_name="c", subcore_axis_name="s")
    k = pl.kernel(
        _copy_body,
        out_type=jax.ShapeDtypeStruct((seq_len, emb), pos_emb_table.dtype),
        scratch_types=[
            pltpu.VMEM((_NBUF, _CHUNK, emb), pos_emb_table.dtype),
            pltpu.SemaphoreType.DMA((_NBUF,)),
            pltpu.SemaphoreType.DMA((_NBUF,)),
        ],
        mesh=mesh,
    )
    return k(pos_emb_table)


# speedup vs baseline: 1.0317x; 1.0317x over previous
import functools

import jax
import jax.numpy as jnp
from jax import lax
from jax.experimental import pallas as pl
from jax.experimental.pallas import tpu as pltpu
from jax.experimental.pallas import tpu_sc as plsc

_NC = 2
_NS = 16
_NW = _NC * _NS
_NBUF = 3


def _copy_body(starts, sizes, table_hbm, out_hbm, buf, in_sems, out_sems):
    wid = lax.axis_index("s") * _NC + lax.axis_index("c")
    rows = out_hbm.shape[0] // _NW
    base = wid * rows
    nchunk = len(sizes)

    def in_copy(j, slot):
        return pltpu.make_async_copy(
            table_hbm.at[pl.ds(base + starts[j], sizes[j])],
            buf.at[slot, pl.ds(0, sizes[j])], in_sems.at[slot])

    def out_copy(j, slot):
        return pltpu.make_async_copy(
            buf.at[slot, pl.ds(0, sizes[j])],
            out_hbm.at[pl.ds(base + starts[j], sizes[j])], out_sems.at[slot])

    for s in range(min(_NBUF, nchunk)):
        in_copy(s, s).start()
    for j in range(nchunk):
        slot = j % _NBUF
        in_copy(j, slot).wait()
        out_copy(j, slot).start()
        out_copy(j, slot).wait()
        if j + _NBUF < nchunk:
            in_copy(j + _NBUF, slot).start()


def kernel(input_ids, pos_emb_table):
    seq_len = input_ids.shape[-1]
    emb = pos_emb_table.shape[1]
    rows = seq_len // _NW
    big = 40
    sizes = []
    left = rows
    while left > 0:
        step = min(big, left)
        sizes.append(step)
        left -= step
    starts = [sum(sizes[:i]) for i in range(len(sizes))]
    mesh = plsc.VectorSubcoreMesh(core_axis_name="c", subcore_axis_name="s")
    k = pl.kernel(
        functools.partial(_copy_body, tuple(starts), tuple(sizes)),
        out_type=jax.ShapeDtypeStruct((seq_len, emb), pos_emb_table.dtype),
        scratch_types=[
            pltpu.VMEM((_NBUF, big, emb), pos_emb_table.dtype),
            pltpu.SemaphoreType.DMA((_NBUF,)),
            pltpu.SemaphoreType.DMA((_NBUF,)),
        ],
        mesh=mesh,
    )
    return k(pos_emb_table)


# speedup vs baseline: 1.0327x; 1.0010x over previous
import functools

import jax
from jax import lax
from jax.experimental import pallas as pl
from jax.experimental.pallas import tpu as pltpu
from jax.experimental.pallas import tpu_sc as plsc

_NC = 2
_NS = 16
_NW = _NC * _NS
_NBUF = 3


def _copy_body(starts, sizes, table_hbm, out_hbm, buf, in_sems, out_sems):
    wid = lax.axis_index("s") * _NC + lax.axis_index("c")
    rows = out_hbm.shape[0] // _NW
    base = wid * rows
    nchunk = len(sizes)

    def in_copy(j, slot):
        return pltpu.make_async_copy(
            table_hbm.at[pl.ds(base + starts[j], sizes[j])],
            buf.at[slot, pl.ds(0, sizes[j])], in_sems.at[slot])

    def out_copy(j, slot):
        return pltpu.make_async_copy(
            buf.at[slot, pl.ds(0, sizes[j])],
            out_hbm.at[pl.ds(base + starts[j], sizes[j])], out_sems.at[slot])

    for s in range(min(_NBUF, nchunk)):
        in_copy(s, s).start()
    for j in range(nchunk):
        slot = j % _NBUF
        in_copy(j, slot).wait()
        out_copy(j, slot).start()
        out_copy(j, slot).wait()
        if j + _NBUF < nchunk:
            in_copy(j + _NBUF, slot).start()


def kernel(input_ids, pos_emb_table):
    seq_len = input_ids.shape[-1]
    emb = pos_emb_table.shape[1]
    rows = seq_len // _NW
    big = 40
    sizes = []
    left = rows
    while left > 0:
        step = min(big, left)
        sizes.append(step)
        left -= step
    starts = [sum(sizes[:i]) for i in range(len(sizes))]
    mesh = plsc.VectorSubcoreMesh(core_axis_name="c", subcore_axis_name="s")
    k = pl.kernel(
        functools.partial(_copy_body, tuple(starts), tuple(sizes)),
        out_type=jax.ShapeDtypeStruct((seq_len, emb), pos_emb_table.dtype),
        scratch_types=[
            pltpu.VMEM((_NBUF, big, emb), pos_emb_table.dtype),
            pltpu.SemaphoreType.DMA((_NBUF,)),
            pltpu.SemaphoreType.DMA((_NBUF,)),
        ],
        mesh=mesh,
    )
    return k(pos_emb_table)
